# baseline (speedup 1.0000x reference)
import jax
import jax.numpy as jnp
from jax.experimental import pallas as pl
from jax.experimental.pallas import tpu as pltpu

DIM = 96
QK = 96
NWIN = 7
HEADS = 8
TOPK = 4
KVW = 4
WH = 32
P2 = NWIN * NWIN
PIX = WH * WH
SPIX = WH * 224
CKV = QK + DIM
DH = QK // HEADS
KVSEL = TOPK * KVW * KVW
KSTK = HEADS * KVSEL
SCALE = QK ** (-0.5)


def _kv_kernel(x_ref, wkv_ref, bkv_ref, kvp_ref):
    xt = jnp.transpose(x_ref[0], (1, 2, 0)).reshape(SPIX, DIM)
    kv = jnp.dot(xt, wkv_ref[...], preferred_element_type=jnp.float32)
    kv = kv + bkv_ref[...]
    kvr = kv.reshape(KVW, 8, NWIN, KVW, 8, CKV)
    kvp_ref[0, 0] = kvr.mean(axis=(1, 4))


def _attn_kernel(ridx_ref, xp_ref, xc_ref, xn_ref, *rest):
    kv_refs = rest[:NWIN * TOPK]
    (wq_ref, bq_ref, wv_ref, bv_ref, wl_ref, wo_ref, bo_ref,
     mask_ref, g_ref, h_ref, o_ref) = rest[NWIN * TOPK:]
    s = pl.program_id(1)

    xc = xc_ref[0]
    q = jnp.dot(xc.reshape(SPIX, DIM), wq_ref[...],
                preferred_element_type=jnp.float32)
    q = (q + bq_ref[...]) * SCALE
    qs = q.reshape(WH, NWIN, WH, QK)

    top = jnp.where(s == 0, 0.0, xp_ref[0, 0])
    bot = jnp.where(s == NWIN - 1, 0.0, xn_ref[0, 0])
    xe = jnp.concatenate([top[None], xc, bot[None]], axis=0)
    v = jnp.dot(xe.reshape((WH + 2) * 224, DIM), wv_ref[...],
                preferred_element_type=jnp.float32)
    v = (v + bv_ref[...]).reshape(WH + 2, 224, DIM)
    zc = jnp.zeros((WH + 2, 1, DIM), jnp.float32)
    ve = jnp.concatenate([zc, v, zc], axis=1)
    lepe = jnp.zeros((WH, 224, DIM), jnp.float32)
    for dy in range(3):
        for dx in range(3):
            lepe = lepe + ve[dy:dy + WH, dx:dx + 224, :] * wl_ref[3 * dy + dx]

    mask = mask_ref[...]
    cols = []
    for pw in range(NWIN):
        qw = qs[:, pw].reshape(PIX, QK)
        blocks = [kv_refs[pw * TOPK + t][0, 0, :, 0, :, :].reshape(KVW * KVW, CKV)
                  for t in range(TOPK)]
        kvsel = jnp.concatenate(blocks, axis=0)
        ksel = kvsel[:, :QK]
        vsel = kvsel[:, QK:]
        kstk = jnp.concatenate([ksel] * HEADS, axis=0) * mask
        vstk = jnp.concatenate([vsel] * HEADS, axis=0) * mask
        sc = jax.lax.dot_general(qw, kstk, (((1,), (1,)), ((), ())),
                                 preferred_element_type=jnp.float32)
        m = jnp.max(sc, axis=1, keepdims=True)
        e = jnp.exp(sc - m)
        o = jnp.dot(e, vstk, preferred_element_type=jnp.float32)
        d = jnp.dot(e, g_ref[...], preferred_element_type=jnp.float32)
        dn = jnp.dot(1.0 / d, h_ref[...], preferred_element_type=jnp.float32)
        attn = o * dn
        z = attn + lepe[:, pw * WH:(pw + 1) * WH, :].reshape(PIX, DIM)
        y = jnp.dot(z, wo_ref[...], preferred_element_type=jnp.float32)
        y = y + bo_ref[...]
        cols.append(y.reshape(WH, WH, DIM))
    o_ref[0] = jnp.transpose(jnp.concatenate(cols, axis=1), (2, 0, 1))


@jax.jit
def kernel(x, W_qkv, b_qkv, W_lepe, W_o, b_o):
    N = x.shape[0]
    f32 = jnp.float32
    xn = jnp.transpose(x, (0, 2, 3, 1))

    kv_pix = pl.pallas_call(
        _kv_kernel,
        grid=(N, NWIN),
        in_specs=[
            pl.BlockSpec((1, DIM, WH, 224), lambda n, s: (n, 0, s, 0)),
            pl.BlockSpec((DIM, CKV), lambda n, s: (0, 0)),
            pl.BlockSpec((1, CKV), lambda n, s: (0, 0)),
        ],
        out_specs=pl.BlockSpec((1, 1, KVW, NWIN, KVW, CKV),
                               lambda n, s: (n, s, 0, 0, 0, 0)),
        out_shape=jax.ShapeDtypeStruct((N, NWIN, KVW, NWIN, KVW, CKV), f32),
    )(x, W_qkv[:, QK:], b_qkv[QK:].reshape(1, CKV))

    xw = xn.reshape(N, NWIN, WH, NWIN, WH, DIM)
    xw = xw.transpose(0, 1, 3, 2, 4, 5).reshape(N, P2, WH, WH, DIM)
    qkv_r = xw @ W_qkv + b_qkv
    q_win = qkv_r[..., :QK].mean(axis=(2, 3))
    k_win = qkv_r[..., QK:2 * QK].mean(axis=(2, 3))
    attn_logit = (q_win * SCALE) @ jnp.swapaxes(k_win, -2, -1)
    _, r_idx = jax.lax.top_k(attn_logit, TOPK)

    head_of_col = jnp.arange(QK, dtype=jnp.int32) // DH
    row_head = jnp.arange(KSTK, dtype=jnp.int32) // KVSEL
    mask = (row_head[:, None] == head_of_col[None, :]).astype(f32)
    g = (row_head[:, None] == jnp.arange(HEADS)[None, :]).astype(f32)
    h = (jnp.arange(HEADS)[:, None] == head_of_col[None, :]).astype(f32)
    wl = jnp.transpose(W_lepe[:, 0], (1, 2, 0)).reshape(9, DIM)

    def kv_imap(pw, t):
        def imap(n, s, r):
            w = r[n, s * NWIN + pw, t]
            return (n, w // NWIN, 0, w % NWIN, 0, 0)
        return imap

    kv_specs = [pl.BlockSpec((1, 1, KVW, 1, KVW, CKV), kv_imap(pw, t))
                for pw in range(NWIN) for t in range(TOPK)]

    y = pl.pallas_call(
        _attn_kernel,
        grid_spec=pltpu.PrefetchScalarGridSpec(
            num_scalar_prefetch=1,
            grid=(N, NWIN),
            in_specs=[
                pl.BlockSpec((1, 1, 224, DIM),
                             lambda n, s, r: (n, jnp.maximum(s * WH - 1, 0), 0, 0)),
                pl.BlockSpec((1, WH, 224, DIM), lambda n, s, r: (n, s, 0, 0)),
                pl.BlockSpec((1, 1, 224, DIM),
                             lambda n, s, r: (n, jnp.minimum(s * WH + WH, 223), 0, 0)),
                *kv_specs,
                pl.BlockSpec((DIM, QK), lambda n, s, r: (0, 0)),
                pl.BlockSpec((1, QK), lambda n, s, r: (0, 0)),
                pl.BlockSpec((DIM, DIM), lambda n, s, r: (0, 0)),
                pl.BlockSpec((1, DIM), lambda n, s, r: (0, 0)),
                pl.BlockSpec((9, DIM), lambda n, s, r: (0, 0)),
                pl.BlockSpec((DIM, DIM), lambda n, s, r: (0, 0)),
                pl.BlockSpec((1, DIM), lambda n, s, r: (0, 0)),
                pl.BlockSpec((KSTK, QK), lambda n, s, r: (0, 0)),
                pl.BlockSpec((KSTK, HEADS), lambda n, s, r: (0, 0)),
                pl.BlockSpec((HEADS, DIM), lambda n, s, r: (0, 0)),
            ],
            out_specs=pl.BlockSpec((1, DIM, WH, 224), lambda n, s, r: (n, 0, s, 0)),
        ),
        out_shape=jax.ShapeDtypeStruct((N, DIM, 224, 224), f32),
    )(r_idx, xn, xn, xn, *([kv_pix] * (NWIN * TOPK)),
      W_qkv[:, :QK], b_qkv[:QK].reshape(1, QK),
      W_qkv[:, 2 * QK:], b_qkv[2 * QK:].reshape(1, DIM),
      wl, W_o, b_o.reshape(1, DIM), mask, g, h)

    return y


# speedup vs baseline: 1.1723x; 1.1723x over previous
import jax
import jax.numpy as jnp
from jax.experimental import pallas as pl
from jax.experimental.pallas import tpu as pltpu

DIM = 96
QK = 96
NWIN = 7
HEADS = 8
TOPK = 4
KVW = 4
WH = 32
P2 = NWIN * NWIN
PIX = WH * WH
SPIX = WH * 224
CKV = QK + DIM
DH = QK // HEADS
KVSEL = TOPK * KVW * KVW
KSTK = HEADS * KVSEL
SCALE = QK ** (-0.5)


def _kv_kernel(x_ref, wkv_ref, bkv_ref, kvp_ref, xn_ref):
    xt3 = jnp.transpose(x_ref[0], (1, 2, 0))
    xn_ref[0] = xt3
    xt = xt3.reshape(SPIX, DIM)
    kv = jnp.dot(xt, wkv_ref[...], preferred_element_type=jnp.float32)
    kv = kv + bkv_ref[...]
    kvr = kv.reshape(KVW, 8, NWIN, KVW, 8, CKV)
    kvp_ref[0, 0] = kvr.mean(axis=(1, 4))


def _attn_kernel(ridx_ref, xp_ref, xc_ref, xn_ref, *rest):
    kv_refs = rest[:NWIN * TOPK]
    (wq_ref, bq_ref, wv_ref, bv_ref, wl_ref, wo_ref, bo_ref,
     mask_ref, g_ref, h_ref, o_ref) = rest[NWIN * TOPK:]
    s = pl.program_id(1)

    xc = xc_ref[0]
    q = jnp.dot(xc.reshape(SPIX, DIM), wq_ref[...],
                preferred_element_type=jnp.float32)
    q = (q + bq_ref[...]) * SCALE
    qs = q.reshape(WH, NWIN, WH, QK)

    top = jnp.where(s == 0, 0.0, xp_ref[0, 0])
    bot = jnp.where(s == NWIN - 1, 0.0, xn_ref[0, 0])
    xe = jnp.concatenate([top[None], xc, bot[None]], axis=0)
    v = jnp.dot(xe.reshape((WH + 2) * 224, DIM), wv_ref[...],
                preferred_element_type=jnp.float32)
    v = (v + bv_ref[...]).reshape(WH + 2, 224, DIM)
    zc = jnp.zeros((WH + 2, 1, DIM), jnp.float32)
    ve = jnp.concatenate([zc, v, zc], axis=1)
    lepe = jnp.zeros((WH, 224, DIM), jnp.float32)
    for dy in range(3):
        for dx in range(3):
            lepe = lepe + ve[dy:dy + WH, dx:dx + 224, :] * wl_ref[3 * dy + dx]

    mask = mask_ref[...]
    cols = []
    for pw in range(NWIN):
        qw = qs[:, pw].reshape(PIX, QK)
        blocks = [kv_refs[pw * TOPK + t][0, 0, :, 0, :, :].reshape(KVW * KVW, CKV)
                  for t in range(TOPK)]
        kvsel = jnp.concatenate(blocks, axis=0)
        ksel = kvsel[:, :QK]
        vsel = kvsel[:, QK:]
        kstk = jnp.concatenate([ksel] * HEADS, axis=0) * mask
        vstk = jnp.concatenate([vsel] * HEADS, axis=0) * mask
        sc = jax.lax.dot_general(qw, kstk, (((1,), (1,)), ((), ())),
                                 preferred_element_type=jnp.float32)
        m = jnp.max(sc, axis=1, keepdims=True)
        e = jnp.exp(sc - m)
        o = jnp.dot(e, vstk, preferred_element_type=jnp.float32)
        d = jnp.dot(e, g_ref[...], preferred_element_type=jnp.float32)
        dn = jnp.dot(1.0 / d, h_ref[...], preferred_element_type=jnp.float32)
        attn = o * dn
        z = attn + lepe[:, pw * WH:(pw + 1) * WH, :].reshape(PIX, DIM)
        y = jnp.dot(z, wo_ref[...], preferred_element_type=jnp.float32)
        y = y + bo_ref[...]
        cols.append(y.reshape(WH, WH, DIM))
    o_ref[0] = jnp.transpose(jnp.concatenate(cols, axis=1), (2, 0, 1))


@jax.jit
def kernel(x, W_qkv, b_qkv, W_lepe, W_o, b_o):
    N = x.shape[0]
    f32 = jnp.float32
    kv_pix, xn = pl.pallas_call(
        _kv_kernel,
        grid=(N, NWIN),
        in_specs=[
            pl.BlockSpec((1, DIM, WH, 224), lambda n, s: (n, 0, s, 0)),
            pl.BlockSpec((DIM, CKV), lambda n, s: (0, 0)),
            pl.BlockSpec((1, CKV), lambda n, s: (0, 0)),
        ],
        out_specs=[
            pl.BlockSpec((1, 1, KVW, NWIN, KVW, CKV),
                         lambda n, s: (n, s, 0, 0, 0, 0)),
            pl.BlockSpec((1, WH, 224, DIM), lambda n, s: (n, s, 0, 0)),
        ],
        out_shape=[
            jax.ShapeDtypeStruct((N, NWIN, KVW, NWIN, KVW, CKV), f32),
            jax.ShapeDtypeStruct((N, 224, 224, DIM), f32),
        ],
    )(x, W_qkv[:, QK:], b_qkv[QK:].reshape(1, CKV))

    xw = xn.reshape(N, NWIN, WH, NWIN, WH, DIM)
    xw = xw.transpose(0, 1, 3, 2, 4, 5).reshape(N, P2, WH, WH, DIM)
    qkv_r = xw @ W_qkv + b_qkv
    q_win = qkv_r[..., :QK].mean(axis=(2, 3))
    k_win = qkv_r[..., QK:2 * QK].mean(axis=(2, 3))
    attn_logit = (q_win * SCALE) @ jnp.swapaxes(k_win, -2, -1)
    _, r_idx = jax.lax.top_k(attn_logit, TOPK)

    head_of_col = jnp.arange(QK, dtype=jnp.int32) // DH
    row_head = jnp.arange(KSTK, dtype=jnp.int32) // KVSEL
    mask = (row_head[:, None] == head_of_col[None, :]).astype(f32)
    g = (row_head[:, None] == jnp.arange(HEADS)[None, :]).astype(f32)
    h = (jnp.arange(HEADS)[:, None] == head_of_col[None, :]).astype(f32)
    wl = jnp.transpose(W_lepe[:, 0], (1, 2, 0)).reshape(9, DIM)

    def kv_imap(pw, t):
        def imap(n, s, r):
            w = r[n, s * NWIN + pw, t]
            return (n, w // NWIN, 0, w % NWIN, 0, 0)
        return imap

    kv_specs = [pl.BlockSpec((1, 1, KVW, 1, KVW, CKV), kv_imap(pw, t))
                for pw in range(NWIN) for t in range(TOPK)]

    y = pl.pallas_call(
        _attn_kernel,
        grid_spec=pltpu.PrefetchScalarGridSpec(
            num_scalar_prefetch=1,
            grid=(N, NWIN),
            in_specs=[
                pl.BlockSpec((1, 1, 224, DIM),
                             lambda n, s, r: (n, jnp.maximum(s * WH - 1, 0), 0, 0)),
                pl.BlockSpec((1, WH, 224, DIM), lambda n, s, r: (n, s, 0, 0)),
                pl.BlockSpec((1, 1, 224, DIM),
                             lambda n, s, r: (n, jnp.minimum(s * WH + WH, 223), 0, 0)),
                *kv_specs,
                pl.BlockSpec((DIM, QK), lambda n, s, r: (0, 0)),
                pl.BlockSpec((1, QK), lambda n, s, r: (0, 0)),
                pl.BlockSpec((DIM, DIM), lambda n, s, r: (0, 0)),
                pl.BlockSpec((1, DIM), lambda n, s, r: (0, 0)),
                pl.BlockSpec((9, DIM), lambda n, s, r: (0, 0)),
                pl.BlockSpec((DIM, DIM), lambda n, s, r: (0, 0)),
                pl.BlockSpec((1, DIM), lambda n, s, r: (0, 0)),
                pl.BlockSpec((KSTK, QK), lambda n, s, r: (0, 0)),
                pl.BlockSpec((KSTK, HEADS), lambda n, s, r: (0, 0)),
                pl.BlockSpec((HEADS, DIM), lambda n, s, r: (0, 0)),
            ],
            out_specs=pl.BlockSpec((1, DIM, WH, 224), lambda n, s, r: (n, 0, s, 0)),
        ),
        out_shape=jax.ShapeDtypeStruct((N, DIM, 224, 224), f32),
    )(r_idx, xn, xn, xn, *([kv_pix] * (NWIN * TOPK)),
      W_qkv[:, :QK], b_qkv[:QK].reshape(1, QK),
      W_qkv[:, 2 * QK:], b_qkv[2 * QK:].reshape(1, DIM),
      wl, W_o, b_o.reshape(1, DIM), mask, g, h)

    return y
